# baseline (speedup 1.0000x reference)
import functools

import jax
import jax.numpy as jnp
from jax import lax
from jax.experimental import pallas as pl
from jax.experimental.pallas import tpu as pltpu
from jax.experimental.pallas import tpu_sc as plsc

_NUM_CORES = 2
_NUM_SUBCORES = 16
_NUM_WORKERS = _NUM_CORES * _NUM_SUBCORES

_CHUNK = 1024
_NSTREAM = 4
_SUB = _CHUNK // _NSTREAM


def _fire_gather(table_hbm, idx_v, rows_v, sem):
    for s in range(_NSTREAM):
        pltpu.async_copy(
            table_hbm.at[idx_v.at[pl.ds(s * _SUB, _SUB)]],
            rows_v.at[pl.ds(s * _SUB, _SUB)], sem)


def _drain_gather(table_hbm, idx_v, rows_v, sem):
    for s in range(_NSTREAM):
        pltpu.make_async_copy(
            table_hbm.at[idx_v.at[pl.ds(s * _SUB, _SUB)]],
            rows_v.at[pl.ds(s * _SUB, _SUB)], sem).wait()


def _gather_kernel(n_rows, d, table_hbm, idx_hbm, out_hbm,
                   idx0, idx1, rows0, rows1, g0, g1, w0, w1):
    rows_per_worker = n_rows // _NUM_WORKERS
    n_chunks = rows_per_worker // _CHUNK
    n_pairs = n_chunks // 2
    wid = lax.axis_index("s") * _NUM_CORES + lax.axis_index("c")
    worker_base = wid * rows_per_worker

    pltpu.sync_copy(idx_hbm.at[pl.ds(worker_base, _CHUNK)], idx0)
    _fire_gather(table_hbm, idx0, rows0, g0)

    def body(i, carry):
        base0 = worker_base + (2 * i) * _CHUNK
        base1 = base0 + _CHUNK
        base2 = base1 + _CHUNK

        pltpu.sync_copy(idx_hbm.at[pl.ds(base1, _CHUNK)], idx1)

        @pl.when(i > 0)
        def _():
            pltpu.make_async_copy(
                rows1, out_hbm.at[pl.ds(base1 - 2 * _CHUNK, _CHUNK)], w1).wait()

        _fire_gather(table_hbm, idx1, rows1, g1)

        _drain_gather(table_hbm, idx0, rows0, g0)
        pltpu.async_copy(rows0, out_hbm.at[pl.ds(base0, _CHUNK)], w0)

        @pl.when(i < n_pairs - 1)
        def _():
            pltpu.sync_copy(idx_hbm.at[pl.ds(base2, _CHUNK)], idx0)

        pltpu.make_async_copy(rows0, out_hbm.at[pl.ds(base0, _CHUNK)], w0).wait()

        @pl.when(i < n_pairs - 1)
        def _():
            _fire_gather(table_hbm, idx0, rows0, g0)

        _drain_gather(table_hbm, idx1, rows1, g1)
        pltpu.async_copy(rows1, out_hbm.at[pl.ds(base1, _CHUNK)], w1)
        return carry

    lax.fori_loop(0, n_pairs, body, 0, unroll=False)

    last_base = worker_base + (n_chunks - 1) * _CHUNK
    pltpu.make_async_copy(rows1, out_hbm.at[pl.ds(last_base, _CHUNK)], w1).wait()


def _sc_gather(a, idx_flat):
    n_rows = idx_flat.shape[0]
    d = a.shape[1]
    mesh = plsc.VectorSubcoreMesh(
        core_axis_name="c", subcore_axis_name="s",
        num_cores=_NUM_CORES, num_subcores=_NUM_SUBCORES,
    )
    k = pl.kernel(
        functools.partial(_gather_kernel, n_rows, d),
        out_type=jax.ShapeDtypeStruct((n_rows, d), jnp.float32),
        mesh=mesh,
        scratch_types=[
            pltpu.VMEM((_CHUNK,), jnp.int32),
            pltpu.VMEM((_CHUNK,), jnp.int32),
            pltpu.VMEM((_CHUNK, d), jnp.float32),
            pltpu.VMEM((_CHUNK, d), jnp.float32),
            pltpu.SemaphoreType.DMA,
            pltpu.SemaphoreType.DMA,
            pltpu.SemaphoreType.DMA,
            pltpu.SemaphoreType.DMA,
        ],
        compiler_params=pltpu.CompilerParams(use_tc_tiling_on_sc=False),
    )
    return k(a, idx_flat)


def _retile_kernel(in_ref, out_ref):
    for gg in range(4):
        wt = in_ref[pl.ds(gg * 128, 128), :].T
        for p in range(4):
            out_ref[0, :, gg * 4 + p, :, :] = (
                wt[p * 32:(p + 1) * 32, :].reshape(4, 8, 128))


def _tc_retile(y):
    return pl.pallas_call(
        _retile_kernel,
        grid=(200, 8),
        in_specs=[pl.BlockSpec((512, 128), lambda j, g: (j * 8 + g, 0))],
        out_specs=pl.BlockSpec(
            (1, 4, 16, 8, 128), lambda j, g: (j, 0, g, 0, 0)),
        out_shape=jax.ShapeDtypeStruct((200, 4, 128, 8, 128), jnp.float32),
    )(y)


def kernel(a, idx):
    d = a.shape[1]
    idx_flat = (idx.T.reshape(200, 32, 4, 128)
                .transpose(0, 1, 3, 2).reshape(-1).astype(jnp.int32))
    outl = _sc_gather(a, idx_flat)
    y = outl.reshape(outl.shape[0] // 4, 4 * d)
    outp = _tc_retile(y)
    out = outp.transpose(2, 4, 0, 1, 3)
    return out.reshape(idx.shape + (d,))


# speedup vs baseline: 1.2653x; 1.2653x over previous
import functools

import jax
import jax.numpy as jnp
from jax import lax
from jax.experimental import pallas as pl
from jax.experimental.pallas import tpu as pltpu
from jax.experimental.pallas import tpu_sc as plsc

_NUM_CORES = 2
_NUM_SUBCORES = 16
_NUM_WORKERS = _NUM_CORES * _NUM_SUBCORES

_CHUNK = 1024
_NSTREAM = 4
_SUB = _CHUNK // _NSTREAM


def _fire_gather(table_hbm, idx_v, rows_v, sem):
    for s in range(_NSTREAM):
        pltpu.async_copy(
            table_hbm.at[idx_v.at[pl.ds(s * _SUB, _SUB)]],
            rows_v.at[pl.ds(s * _SUB, _SUB)], sem)


def _drain_gather(table_hbm, idx_v, rows_v, sem):
    for s in range(_NSTREAM):
        pltpu.make_async_copy(
            table_hbm.at[idx_v.at[pl.ds(s * _SUB, _SUB)]],
            rows_v.at[pl.ds(s * _SUB, _SUB)], sem).wait()


def _gather_kernel(n_rows, d, table_hbm, idx_hbm, out_hbm,
                   idx0, idx1, rows0, rows1, g0, g1, w0, w1):
    rows_per_worker = n_rows // _NUM_WORKERS
    n_chunks = rows_per_worker // _CHUNK
    n_pairs = n_chunks // 2
    wid = lax.axis_index("s") * _NUM_CORES + lax.axis_index("c")
    worker_base = wid * rows_per_worker

    pltpu.sync_copy(idx_hbm.at[pl.ds(worker_base, _CHUNK)], idx0)
    _fire_gather(table_hbm, idx0, rows0, g0)

    def body(i, carry):
        base0 = worker_base + (2 * i) * _CHUNK
        base1 = base0 + _CHUNK
        base2 = base1 + _CHUNK

        pltpu.sync_copy(idx_hbm.at[pl.ds(base1, _CHUNK)], idx1)

        @pl.when(i > 0)
        def _():
            pltpu.make_async_copy(
                rows1, out_hbm.at[pl.ds(base1 - 2 * _CHUNK, _CHUNK)], w1).wait()

        _fire_gather(table_hbm, idx1, rows1, g1)

        _drain_gather(table_hbm, idx0, rows0, g0)
        pltpu.async_copy(rows0, out_hbm.at[pl.ds(base0, _CHUNK)], w0)

        @pl.when(i < n_pairs - 1)
        def _():
            pltpu.sync_copy(idx_hbm.at[pl.ds(base2, _CHUNK)], idx0)

        pltpu.make_async_copy(rows0, out_hbm.at[pl.ds(base0, _CHUNK)], w0).wait()

        @pl.when(i < n_pairs - 1)
        def _():
            _fire_gather(table_hbm, idx0, rows0, g0)

        _drain_gather(table_hbm, idx1, rows1, g1)
        pltpu.async_copy(rows1, out_hbm.at[pl.ds(base1, _CHUNK)], w1)
        return carry

    lax.fori_loop(0, n_pairs, body, 0, unroll=False)

    last_base = worker_base + (n_chunks - 1) * _CHUNK
    pltpu.make_async_copy(rows1, out_hbm.at[pl.ds(last_base, _CHUNK)], w1).wait()


def _sc_gather(a, idx_flat):
    n_rows = idx_flat.shape[0]
    d = a.shape[1]
    mesh = plsc.VectorSubcoreMesh(
        core_axis_name="c", subcore_axis_name="s",
        num_cores=_NUM_CORES, num_subcores=_NUM_SUBCORES,
    )
    k = pl.kernel(
        functools.partial(_gather_kernel, n_rows, d),
        out_type=jax.ShapeDtypeStruct((n_rows, d), jnp.float32),
        mesh=mesh,
        scratch_types=[
            pltpu.VMEM((_CHUNK,), jnp.int32),
            pltpu.VMEM((_CHUNK,), jnp.int32),
            pltpu.VMEM((_CHUNK, d), jnp.float32),
            pltpu.VMEM((_CHUNK, d), jnp.float32),
            pltpu.SemaphoreType.DMA,
            pltpu.SemaphoreType.DMA,
            pltpu.SemaphoreType.DMA,
            pltpu.SemaphoreType.DMA,
        ],
        compiler_params=pltpu.CompilerParams(use_tc_tiling_on_sc=False),
    )
    return k(a, idx_flat)


def _retile_kernel(in_ref, out_ref):
    for gg in range(16):
        wt = in_ref[pl.ds(gg * 128, 128), :].T
        for p in range(4):
            out_ref[0, :, gg * 4 + p, :, :] = (
                wt[p * 32:(p + 1) * 32, :].reshape(4, 8, 128))


def _tc_retile(y):
    return pl.pallas_call(
        _retile_kernel,
        grid=(200, 2),
        in_specs=[pl.BlockSpec((2048, 128), lambda j, g: (j * 2 + g, 0))],
        out_specs=pl.BlockSpec(
            (1, 4, 64, 8, 128), lambda j, g: (j, 0, g, 0, 0)),
        out_shape=jax.ShapeDtypeStruct((200, 4, 128, 8, 128), jnp.float32),
    )(y)


def kernel(a, idx):
    d = a.shape[1]
    idx_flat = (idx.T.reshape(200, 32, 4, 128)
                .transpose(0, 1, 3, 2).reshape(-1).astype(jnp.int32))
    outl = _sc_gather(a, idx_flat)
    y = outl.reshape(outl.shape[0] // 4, 4 * d)
    outp = _tc_retile(y)
    out = outp.transpose(2, 4, 0, 1, 3)
    return out.reshape(idx.shape + (d,))


# speedup vs baseline: 1.3397x; 1.0588x over previous
import functools

import jax
import jax.numpy as jnp
from jax import lax
from jax.experimental import pallas as pl
from jax.experimental.pallas import tpu as pltpu
from jax.experimental.pallas import tpu_sc as plsc

_NUM_CORES = 2
_NUM_SUBCORES = 16
_NUM_WORKERS = _NUM_CORES * _NUM_SUBCORES

_CHUNK = 1024
_NSTREAM = 4
_SUB = _CHUNK // _NSTREAM


def _fire_gather(table_hbm, idx_v, rows_v, sem):
    for s in range(_NSTREAM):
        pltpu.async_copy(
            table_hbm.at[idx_v.at[pl.ds(s * _SUB, _SUB)]],
            rows_v.at[pl.ds(s * _SUB, _SUB)], sem)


def _drain_gather(table_hbm, idx_v, rows_v, sem):
    for s in range(_NSTREAM):
        pltpu.make_async_copy(
            table_hbm.at[idx_v.at[pl.ds(s * _SUB, _SUB)]],
            rows_v.at[pl.ds(s * _SUB, _SUB)], sem).wait()


def _gather_kernel(n_rows, d, table_hbm, idx_hbm, out_hbm,
                   idx0, idx1, rows0, rows1, g0, g1, w0, w1):
    rows_per_worker = n_rows // _NUM_WORKERS
    n_chunks = rows_per_worker // _CHUNK
    n_pairs = n_chunks // 2
    wid = lax.axis_index("s") * _NUM_CORES + lax.axis_index("c")
    worker_base = wid * rows_per_worker

    pltpu.sync_copy(idx_hbm.at[pl.ds(worker_base, _CHUNK)], idx0)
    _fire_gather(table_hbm, idx0, rows0, g0)

    def body(i, carry):
        base0 = worker_base + (2 * i) * _CHUNK
        base1 = base0 + _CHUNK
        base2 = base1 + _CHUNK

        pltpu.sync_copy(idx_hbm.at[pl.ds(base1, _CHUNK)], idx1)

        @pl.when(i > 0)
        def _():
            pltpu.make_async_copy(
                rows1, out_hbm.at[pl.ds(base1 - 2 * _CHUNK, _CHUNK)], w1).wait()

        _fire_gather(table_hbm, idx1, rows1, g1)

        _drain_gather(table_hbm, idx0, rows0, g0)
        pltpu.async_copy(rows0, out_hbm.at[pl.ds(base0, _CHUNK)], w0)

        @pl.when(i < n_pairs - 1)
        def _():
            pltpu.sync_copy(idx_hbm.at[pl.ds(base2, _CHUNK)], idx0)

        pltpu.make_async_copy(rows0, out_hbm.at[pl.ds(base0, _CHUNK)], w0).wait()

        @pl.when(i < n_pairs - 1)
        def _():
            _fire_gather(table_hbm, idx0, rows0, g0)

        _drain_gather(table_hbm, idx1, rows1, g1)
        pltpu.async_copy(rows1, out_hbm.at[pl.ds(base1, _CHUNK)], w1)
        return carry

    lax.fori_loop(0, n_pairs, body, 0, unroll=False)

    last_base = worker_base + (n_chunks - 1) * _CHUNK
    pltpu.make_async_copy(rows1, out_hbm.at[pl.ds(last_base, _CHUNK)], w1).wait()


def _sc_gather(a, idx_flat):
    n_rows = idx_flat.shape[0]
    d = a.shape[1]
    mesh = plsc.VectorSubcoreMesh(
        core_axis_name="c", subcore_axis_name="s",
        num_cores=_NUM_CORES, num_subcores=_NUM_SUBCORES,
    )
    k = pl.kernel(
        functools.partial(_gather_kernel, n_rows, d),
        out_type=jax.ShapeDtypeStruct((n_rows, d), jnp.float32),
        mesh=mesh,
        scratch_types=[
            pltpu.VMEM((_CHUNK,), jnp.int32),
            pltpu.VMEM((_CHUNK,), jnp.int32),
            pltpu.VMEM((_CHUNK, d), jnp.float32),
            pltpu.VMEM((_CHUNK, d), jnp.float32),
            pltpu.SemaphoreType.DMA,
            pltpu.SemaphoreType.DMA,
            pltpu.SemaphoreType.DMA,
            pltpu.SemaphoreType.DMA,
        ],
        compiler_params=pltpu.CompilerParams(use_tc_tiling_on_sc=False),
    )
    return k(a, idx_flat)


def _retile_kernel(in_ref, out_ref):
    for gg in range(32):
        wt = in_ref[pl.ds(gg * 128, 128), :].T
        for p in range(4):
            out_ref[0, :, gg * 4 + p, :, :] = (
                wt[p * 32:(p + 1) * 32, :].reshape(4, 8, 128))


def _tc_retile(y):
    return pl.pallas_call(
        _retile_kernel,
        grid=(200,),
        in_specs=[pl.BlockSpec((4096, 128), lambda j: (j, 0))],
        out_specs=pl.BlockSpec(
            (1, 4, 128, 8, 128), lambda j: (j, 0, 0, 0, 0)),
        out_shape=jax.ShapeDtypeStruct((200, 4, 128, 8, 128), jnp.float32),
    )(y)


def kernel(a, idx):
    d = a.shape[1]
    idx_flat = (idx.T.reshape(200, 32, 4, 128)
                .transpose(0, 1, 3, 2).reshape(-1).astype(jnp.int32))
    outl = _sc_gather(a, idx_flat)
    y = outl.reshape(outl.shape[0] // 4, 4 * d)
    outp = _tc_retile(y)
    out = outp.transpose(2, 4, 0, 1, 3)
    return out.reshape(idx.shape + (d,))


# speedup vs baseline: 1.3757x; 1.0268x over previous
import functools

import jax
import jax.numpy as jnp
from jax import lax
from jax.experimental import pallas as pl
from jax.experimental.pallas import tpu as pltpu
from jax.experimental.pallas import tpu_sc as plsc

_NUM_CORES = 2
_NUM_SUBCORES = 16
_NUM_WORKERS = _NUM_CORES * _NUM_SUBCORES

_CHUNK = 1024
_NSTREAM = 4
_SUB = _CHUNK // _NSTREAM


def _fire_gather(table_hbm, idx_v, rows_v, sem):
    for s in range(_NSTREAM):
        pltpu.async_copy(
            table_hbm.at[idx_v.at[pl.ds(s * _SUB, _SUB)]],
            rows_v.at[pl.ds(s * _SUB, _SUB)], sem)


def _drain_gather(table_hbm, idx_v, rows_v, sem):
    for s in range(_NSTREAM):
        pltpu.make_async_copy(
            table_hbm.at[idx_v.at[pl.ds(s * _SUB, _SUB)]],
            rows_v.at[pl.ds(s * _SUB, _SUB)], sem).wait()


def _gather_kernel(n_rows, d, table_hbm, idx_hbm, out_hbm,
                   idx0, idx1, rows0, rows1, g0, g1, w0, w1):
    rows_per_worker = n_rows // _NUM_WORKERS
    n_chunks = rows_per_worker // _CHUNK
    n_pairs = n_chunks // 2
    wid = lax.axis_index("s") * _NUM_CORES + lax.axis_index("c")
    worker_base = wid * rows_per_worker

    pltpu.sync_copy(idx_hbm.at[pl.ds(worker_base, _CHUNK)], idx0)
    _fire_gather(table_hbm, idx0, rows0, g0)

    def body(i, carry):
        base0 = worker_base + (2 * i) * _CHUNK
        base1 = base0 + _CHUNK
        base2 = base1 + _CHUNK

        pltpu.sync_copy(idx_hbm.at[pl.ds(base1, _CHUNK)], idx1)

        @pl.when(i > 0)
        def _():
            pltpu.make_async_copy(
                rows1, out_hbm.at[pl.ds(base1 - 2 * _CHUNK, _CHUNK)], w1).wait()

        _fire_gather(table_hbm, idx1, rows1, g1)

        _drain_gather(table_hbm, idx0, rows0, g0)
        pltpu.async_copy(rows0, out_hbm.at[pl.ds(base0, _CHUNK)], w0)

        @pl.when(i < n_pairs - 1)
        def _():
            pltpu.sync_copy(idx_hbm.at[pl.ds(base2, _CHUNK)], idx0)

        pltpu.make_async_copy(rows0, out_hbm.at[pl.ds(base0, _CHUNK)], w0).wait()

        @pl.when(i < n_pairs - 1)
        def _():
            _fire_gather(table_hbm, idx0, rows0, g0)

        _drain_gather(table_hbm, idx1, rows1, g1)
        pltpu.async_copy(rows1, out_hbm.at[pl.ds(base1, _CHUNK)], w1)
        return carry

    lax.fori_loop(0, n_pairs, body, 0, unroll=False)

    last_base = worker_base + (n_chunks - 1) * _CHUNK
    pltpu.make_async_copy(rows1, out_hbm.at[pl.ds(last_base, _CHUNK)], w1).wait()


def _sc_gather(a, idx_flat):
    n_rows = idx_flat.shape[0]
    d = a.shape[1]
    mesh = plsc.VectorSubcoreMesh(
        core_axis_name="c", subcore_axis_name="s",
        num_cores=_NUM_CORES, num_subcores=_NUM_SUBCORES,
    )
    k = pl.kernel(
        functools.partial(_gather_kernel, n_rows, d),
        out_type=jax.ShapeDtypeStruct((n_rows, d), jnp.float32),
        mesh=mesh,
        scratch_types=[
            pltpu.VMEM((_CHUNK,), jnp.int32),
            pltpu.VMEM((_CHUNK,), jnp.int32),
            pltpu.VMEM((_CHUNK, d), jnp.float32),
            pltpu.VMEM((_CHUNK, d), jnp.float32),
            pltpu.SemaphoreType.DMA,
            pltpu.SemaphoreType.DMA,
            pltpu.SemaphoreType.DMA,
            pltpu.SemaphoreType.DMA,
        ],
        compiler_params=pltpu.CompilerParams(use_tc_tiling_on_sc=False),
    )
    return k(a, idx_flat)


def _retile_kernel(in_ref, out_ref):
    for jj in range(4):
        for gg in range(32):
            wt = in_ref[pl.ds(jj * 4096 + gg * 128, 128), :].T
            for p in range(4):
                out_ref[jj, :, gg * 4 + p, :, :] = (
                    wt[p * 32:(p + 1) * 32, :].reshape(4, 8, 128))


def _tc_retile(y):
    return pl.pallas_call(
        _retile_kernel,
        grid=(50,),
        in_specs=[pl.BlockSpec((16384, 128), lambda j: (j, 0))],
        out_specs=pl.BlockSpec(
            (4, 4, 128, 8, 128), lambda j: (j, 0, 0, 0, 0)),
        out_shape=jax.ShapeDtypeStruct((200, 4, 128, 8, 128), jnp.float32),
    )(y)


def kernel(a, idx):
    d = a.shape[1]
    idx_flat = (idx.T.reshape(200, 32, 4, 128)
                .transpose(0, 1, 3, 2).reshape(-1).astype(jnp.int32))
    outl = _sc_gather(a, idx_flat)
    y = outl.reshape(outl.shape[0] // 4, 4 * d)
    outp = _tc_retile(y)
    out = outp.transpose(2, 4, 0, 1, 3)
    return out.reshape(idx.shape + (d,))


# speedup vs baseline: 1.3759x; 1.0002x over previous
import functools

import jax
import jax.numpy as jnp
from jax import lax
from jax.experimental import pallas as pl
from jax.experimental.pallas import tpu as pltpu
from jax.experimental.pallas import tpu_sc as plsc

_NUM_CORES = 2
_NUM_SUBCORES = 16
_NUM_WORKERS = _NUM_CORES * _NUM_SUBCORES

_CHUNK = 1024
_NSTREAM = 4
_SUB = _CHUNK // _NSTREAM


def _fire_gather(table_hbm, idx_v, rows_v, sem):
    for s in range(_NSTREAM):
        pltpu.async_copy(
            table_hbm.at[idx_v.at[pl.ds(s * _SUB, _SUB)]],
            rows_v.at[pl.ds(s * _SUB, _SUB)], sem)


def _drain_gather(table_hbm, idx_v, rows_v, sem):
    for s in range(_NSTREAM):
        pltpu.make_async_copy(
            table_hbm.at[idx_v.at[pl.ds(s * _SUB, _SUB)]],
            rows_v.at[pl.ds(s * _SUB, _SUB)], sem).wait()


def _gather_kernel(n_rows, d, table_hbm, idx_hbm, out_hbm,
                   idx0, idx1, rows0, rows1, g0, g1, w0, w1):
    rows_per_worker = n_rows // _NUM_WORKERS
    n_chunks = rows_per_worker // _CHUNK
    n_pairs = n_chunks // 2
    wid = lax.axis_index("s") * _NUM_CORES + lax.axis_index("c")
    worker_base = wid * rows_per_worker

    pltpu.sync_copy(idx_hbm.at[pl.ds(worker_base, _CHUNK)], idx0)
    _fire_gather(table_hbm, idx0, rows0, g0)

    def body(i, carry):
        base0 = worker_base + (2 * i) * _CHUNK
        base1 = base0 + _CHUNK
        base2 = base1 + _CHUNK

        pltpu.sync_copy(idx_hbm.at[pl.ds(base1, _CHUNK)], idx1)

        @pl.when(i > 0)
        def _():
            pltpu.make_async_copy(
                rows1, out_hbm.at[pl.ds(base1 - 2 * _CHUNK, _CHUNK)], w1).wait()

        _fire_gather(table_hbm, idx1, rows1, g1)

        _drain_gather(table_hbm, idx0, rows0, g0)
        pltpu.async_copy(rows0, out_hbm.at[pl.ds(base0, _CHUNK)], w0)

        @pl.when(i < n_pairs - 1)
        def _():
            pltpu.sync_copy(idx_hbm.at[pl.ds(base2, _CHUNK)], idx0)

        pltpu.make_async_copy(rows0, out_hbm.at[pl.ds(base0, _CHUNK)], w0).wait()

        @pl.when(i < n_pairs - 1)
        def _():
            _fire_gather(table_hbm, idx0, rows0, g0)

        _drain_gather(table_hbm, idx1, rows1, g1)
        pltpu.async_copy(rows1, out_hbm.at[pl.ds(base1, _CHUNK)], w1)
        return carry

    lax.fori_loop(0, n_pairs, body, 0, unroll=False)

    last_base = worker_base + (n_chunks - 1) * _CHUNK
    pltpu.make_async_copy(rows1, out_hbm.at[pl.ds(last_base, _CHUNK)], w1).wait()


def _sc_gather(a, idx_flat):
    n_rows = idx_flat.shape[0]
    d = a.shape[1]
    mesh = plsc.VectorSubcoreMesh(
        core_axis_name="c", subcore_axis_name="s",
        num_cores=_NUM_CORES, num_subcores=_NUM_SUBCORES,
    )
    k = pl.kernel(
        functools.partial(_gather_kernel, n_rows, d),
        out_type=jax.ShapeDtypeStruct((n_rows, d), jnp.float32),
        mesh=mesh,
        scratch_types=[
            pltpu.VMEM((_CHUNK,), jnp.int32),
            pltpu.VMEM((_CHUNK,), jnp.int32),
            pltpu.VMEM((_CHUNK, d), jnp.float32),
            pltpu.VMEM((_CHUNK, d), jnp.float32),
            pltpu.SemaphoreType.DMA,
            pltpu.SemaphoreType.DMA,
            pltpu.SemaphoreType.DMA,
            pltpu.SemaphoreType.DMA,
        ],
        compiler_params=pltpu.CompilerParams(use_tc_tiling_on_sc=False),
    )
    return k(a, idx_flat)


def _retile_kernel(in_ref, out_ref):
    for jj in range(5):
        for gg in range(32):
            wt = in_ref[pl.ds(jj * 4096 + gg * 128, 128), :].T
            for p in range(4):
                out_ref[jj, :, gg * 4 + p, :, :] = (
                    wt[p * 32:(p + 1) * 32, :].reshape(4, 8, 128))


def _tc_retile(y):
    return pl.pallas_call(
        _retile_kernel,
        grid=(40,),
        in_specs=[pl.BlockSpec((20480, 128), lambda j: (j, 0))],
        out_specs=pl.BlockSpec(
            (5, 4, 128, 8, 128), lambda j: (j, 0, 0, 0, 0)),
        out_shape=jax.ShapeDtypeStruct((200, 4, 128, 8, 128), jnp.float32),
    )(y)


def kernel(a, idx):
    d = a.shape[1]
    idx_flat = (idx.T.reshape(200, 32, 4, 128)
                .transpose(0, 1, 3, 2).reshape(-1).astype(jnp.int32))
    outl = _sc_gather(a, idx_flat)
    y = outl.reshape(outl.shape[0] // 4, 4 * d)
    outp = _tc_retile(y)
    out = outp.transpose(2, 4, 0, 1, 3)
    return out.reshape(idx.shape + (d,))


# speedup vs baseline: 1.3762x; 1.0002x over previous
import functools

import jax
import jax.numpy as jnp
from jax import lax
from jax.experimental import pallas as pl
from jax.experimental.pallas import tpu as pltpu
from jax.experimental.pallas import tpu_sc as plsc

_NUM_CORES = 2
_NUM_SUBCORES = 16
_NUM_WORKERS = _NUM_CORES * _NUM_SUBCORES

_CHUNK = 1280
_NSTREAM = 4
_SUB = _CHUNK // _NSTREAM


def _fire_gather(table_hbm, idx_v, rows_v, sem):
    for s in range(_NSTREAM):
        pltpu.async_copy(
            table_hbm.at[idx_v.at[pl.ds(s * _SUB, _SUB)]],
            rows_v.at[pl.ds(s * _SUB, _SUB)], sem)


def _drain_gather(table_hbm, idx_v, rows_v, sem):
    for s in range(_NSTREAM):
        pltpu.make_async_copy(
            table_hbm.at[idx_v.at[pl.ds(s * _SUB, _SUB)]],
            rows_v.at[pl.ds(s * _SUB, _SUB)], sem).wait()


def _gather_kernel(n_rows, d, table_hbm, idx_hbm, out_hbm,
                   idx0, idx1, rows0, rows1, g0, g1, w0, w1):
    rows_per_worker = n_rows // _NUM_WORKERS
    n_chunks = rows_per_worker // _CHUNK
    n_pairs = n_chunks // 2
    wid = lax.axis_index("s") * _NUM_CORES + lax.axis_index("c")
    worker_base = wid * rows_per_worker

    pltpu.sync_copy(idx_hbm.at[pl.ds(worker_base, _CHUNK)], idx0)
    _fire_gather(table_hbm, idx0, rows0, g0)

    def body(i, carry):
        base0 = worker_base + (2 * i) * _CHUNK
        base1 = base0 + _CHUNK
        base2 = base1 + _CHUNK

        pltpu.sync_copy(idx_hbm.at[pl.ds(base1, _CHUNK)], idx1)

        @pl.when(i > 0)
        def _():
            pltpu.make_async_copy(
                rows1, out_hbm.at[pl.ds(base1 - 2 * _CHUNK, _CHUNK)], w1).wait()

        _fire_gather(table_hbm, idx1, rows1, g1)

        _drain_gather(table_hbm, idx0, rows0, g0)
        pltpu.async_copy(rows0, out_hbm.at[pl.ds(base0, _CHUNK)], w0)

        @pl.when(i < n_pairs - 1)
        def _():
            pltpu.sync_copy(idx_hbm.at[pl.ds(base2, _CHUNK)], idx0)

        pltpu.make_async_copy(rows0, out_hbm.at[pl.ds(base0, _CHUNK)], w0).wait()

        @pl.when(i < n_pairs - 1)
        def _():
            _fire_gather(table_hbm, idx0, rows0, g0)

        _drain_gather(table_hbm, idx1, rows1, g1)
        pltpu.async_copy(rows1, out_hbm.at[pl.ds(base1, _CHUNK)], w1)
        return carry

    lax.fori_loop(0, n_pairs, body, 0, unroll=False)

    last_base = worker_base + (n_chunks - 1) * _CHUNK
    pltpu.make_async_copy(rows1, out_hbm.at[pl.ds(last_base, _CHUNK)], w1).wait()


def _sc_gather(a, idx_flat):
    n_rows = idx_flat.shape[0]
    d = a.shape[1]
    mesh = plsc.VectorSubcoreMesh(
        core_axis_name="c", subcore_axis_name="s",
        num_cores=_NUM_CORES, num_subcores=_NUM_SUBCORES,
    )
    k = pl.kernel(
        functools.partial(_gather_kernel, n_rows, d),
        out_type=jax.ShapeDtypeStruct((n_rows, d), jnp.float32),
        mesh=mesh,
        scratch_types=[
            pltpu.VMEM((_CHUNK,), jnp.int32),
            pltpu.VMEM((_CHUNK,), jnp.int32),
            pltpu.VMEM((_CHUNK, d), jnp.float32),
            pltpu.VMEM((_CHUNK, d), jnp.float32),
            pltpu.SemaphoreType.DMA,
            pltpu.SemaphoreType.DMA,
            pltpu.SemaphoreType.DMA,
            pltpu.SemaphoreType.DMA,
        ],
        compiler_params=pltpu.CompilerParams(use_tc_tiling_on_sc=False),
    )
    return k(a, idx_flat)


def _retile_kernel(in_ref, out_ref):
    for jj in range(5):
        for gg in range(32):
            wt = in_ref[pl.ds(jj * 4096 + gg * 128, 128), :].T
            for p in range(4):
                out_ref[jj, :, gg * 4 + p, :, :] = (
                    wt[p * 32:(p + 1) * 32, :].reshape(4, 8, 128))


def _tc_retile(y):
    return pl.pallas_call(
        _retile_kernel,
        grid=(40,),
        in_specs=[pl.BlockSpec((20480, 128), lambda j: (j, 0))],
        out_specs=pl.BlockSpec(
            (5, 4, 128, 8, 128), lambda j: (j, 0, 0, 0, 0)),
        out_shape=jax.ShapeDtypeStruct((200, 4, 128, 8, 128), jnp.float32),
    )(y)


def kernel(a, idx):
    d = a.shape[1]
    idx_flat = (idx.T.reshape(200, 32, 4, 128)
                .transpose(0, 1, 3, 2).reshape(-1).astype(jnp.int32))
    outl = _sc_gather(a, idx_flat)
    y = outl.reshape(outl.shape[0] // 4, 4 * d)
    outp = _tc_retile(y)
    out = outp.transpose(2, 4, 0, 1, 3)
    return out.reshape(idx.shape + (d,))
